# baseline (speedup 1.0000x reference)
import functools

import jax
import jax.numpy as jnp
from jax import lax
from jax.experimental import pallas as pl
from jax.experimental.pallas import tpu as pltpu
from jax.experimental.pallas import tpu_sc as plsc

D_MODEL = 64
NC = 2
NS = 16
NW = NC * NS
BW = 128
LANES = 16


@functools.lru_cache(maxsize=None)
def _make_sc_gather(NB, NH, VM):
    n_units = NB // BW * NH
    u_per_w = n_units // NW
    assert u_per_w % 4 == 0
    half = u_per_w // 2
    mesh = plsc.VectorSubcoreMesh(core_axis_name="c", subcore_axis_name="s")

    @functools.partial(
        pl.kernel,
        mesh=mesh,
        out_type=jax.ShapeDtypeStruct((NH, D_MODEL, NB), jnp.float32),
        scratch_types=[
            pltpu.VMEM_SHARED((VM, 2 * D_MODEL), jnp.float32),
            pltpu.VMEM((half * BW,), jnp.int32),
            pltpu.VMEM((BW, 2 * D_MODEL), jnp.float32),
            pltpu.VMEM((BW, 2 * D_MODEL), jnp.float32),
            pltpu.VMEM((D_MODEL, BW), jnp.float32),
            pltpu.VMEM((D_MODEL, BW), jnp.float32),
            pltpu.SemaphoreType.DMA,
            pltpu.SemaphoreType.DMA,
            pltpu.SemaphoreType.DMA,
            pltpu.SemaphoreType.DMA,
            pltpu.SemaphoreType.DMA,
        ],
        compiler_params=pltpu.CompilerParams(use_tc_tiling_on_sc=True,
                                             needs_layout_passes=False),
    )
    def k(table_hbm, idx_hbm, out_hbm, table_sh, idx_v, bufg0, bufg1,
          buft0, buft1, isem, gsem0, gsem1, osem0, osem1):
        sid = lax.axis_index("s")
        wid = sid * NC + lax.axis_index("c")
        ubase = wid * u_per_w
        bufg = (bufg0, bufg1)
        buft = (buft0, buft1)
        gsem = (gsem0, gsem1)
        osem = (osem0, osem1)

        @pl.when(sid == 0)
        def _():
            pltpu.sync_copy(table_hbm, table_sh)

        plsc.subcore_barrier()

        iota16 = lax.iota(jnp.int32, LANES)
        zeros16 = iota16 - iota16
        rows_g = tuple(iota16 + gg * LANES for gg in range(8))

        def iblk_load(i):
            pltpu.async_copy(
                idx_hbm.at[pl.ds((ubase + i * half) * BW, half * BW)],
                idx_v, isem)

        def iblk_wait():
            pltpu.make_async_copy(idx_hbm.at[pl.ds(0, half * BW)],
                                  idx_v, isem).wait()

        def gather_start(j, sg):
            pltpu.async_copy(
                table_sh.at[idx_v.at[pl.ds(j * BW, BW)]],
                bufg[sg], gsem[sg])

        def gather_wait(sg):
            pltpu.make_async_copy(table_sh.at[idx_v.at[pl.ds(0, BW)]],
                                  bufg[sg], gsem[sg]).wait()

        def transpose(s):
            g = bufg[s]
            t = buft[s]

            @plsc.parallel_loop(0, D_MODEL, step=1, unroll=8)
            def dbody(d):
                col = zeros16 + d
                for gg in range(8):
                    t[d, pl.ds(gg * LANES, LANES)] = plsc.load_gather(
                        g, [rows_g[gg], col])

        def store_start(u, s):
            gu = ubase + u
            h = gu >> 7
            b0 = (gu & (BW - 1)) * BW
            pltpu.async_copy(buft[s], out_hbm.at[h, :, pl.ds(b0, BW)],
                             osem[s])

        def store_wait(s):
            pltpu.make_async_copy(buft[s], out_hbm.at[0, :, pl.ds(0, BW)],
                                  osem[s]).wait()

        def unit(u, sg, jn, first=False, start_next=True):
            gather_wait(sg)
            if start_next:
                gather_start(jn, sg ^ 1)
            if not first:
                store_wait(sg)
            transpose(sg)
            store_start(u, sg)

        iblk_load(0)
        iblk_wait()
        gather_start(0, 0)

        unit(0, 0, 1, first=True)
        unit(1, 1, 2, first=True)

        def body0(m, carry):
            u = 2 * m
            unit(u, 0, u + 1)
            unit(u + 1, 1, u + 2)
            return carry

        lax.fori_loop(1, half // 2 - 1, body0, 0)

        unit(half - 2, 0, half - 1)
        gather_wait(1)
        iblk_load(1)
        store_wait(1)
        transpose(1)
        store_start(half - 1, 1)
        iblk_wait()
        gather_start(0, 0)

        def body1(m, carry):
            u = 2 * m
            unit(half + u, 0, u + 1)
            unit(half + u + 1, 1, u + 2)
            return carry

        lax.fori_loop(0, half // 2 - 1, body1, 0)

        unit(2 * half - 2, 0, half - 1)
        unit(2 * half - 1, 1, 0, start_next=False)

        store_wait(0)
        store_wait(1)

    return k


def kernel(x, pe):
    nb, nh = x.shape
    V = pe.shape[0]
    H = V // 2
    idx = x.astype(jnp.int32)
    pe_sh = jnp.concatenate([pe[1:], jnp.zeros((1, D_MODEL), jnp.float32)], 0)
    table_m = jnp.concatenate([pe.reshape(H, 2 * D_MODEL),
                               pe_sh.reshape(H, 2 * D_MODEL)], 0)
    r = (idx >> 1) + (idx & 1) * H
    rt = r.T.reshape(nb * nh)
    out = _make_sc_gather(nb, nh, V)(table_m, rt)
    return jnp.transpose(out, (2, 0, 1))


# speedup vs baseline: 3.6205x; 3.6205x over previous
import functools

import jax
import jax.numpy as jnp
from jax import lax
from jax.experimental import pallas as pl
from jax.experimental.pallas import tpu as pltpu
from jax.experimental.pallas import tpu_sc as plsc

D_MODEL = 64
NC = 2
NS = 16
NW = NC * NS
BW = 128
LANES = 16


@functools.lru_cache(maxsize=None)
def _make_sc_gather(NB, NH, VM):
    n_units = NB // BW * NH
    u_per_w = n_units // NW
    assert u_per_w % 4 == 0
    half = u_per_w // 2
    mesh = plsc.VectorSubcoreMesh(core_axis_name="c", subcore_axis_name="s")

    @functools.partial(
        pl.kernel,
        mesh=mesh,
        out_type=jax.ShapeDtypeStruct((NH, D_MODEL, NB), jnp.float32),
        scratch_types=[
            pltpu.VMEM_SHARED((VM, 2 * D_MODEL), jnp.float32),
            pltpu.VMEM((half * BW,), jnp.int32),
            pltpu.VMEM((BW, 2 * D_MODEL), jnp.float32),
            pltpu.VMEM((BW, 2 * D_MODEL), jnp.float32),
            pltpu.VMEM((D_MODEL, BW), jnp.float32),
            pltpu.VMEM((D_MODEL, BW), jnp.float32),
            pltpu.SemaphoreType.DMA,
            pltpu.SemaphoreType.DMA,
            pltpu.SemaphoreType.DMA,
            pltpu.SemaphoreType.DMA,
            pltpu.SemaphoreType.DMA,
        ],
        compiler_params=pltpu.CompilerParams(use_tc_tiling_on_sc=True,
                                             needs_layout_passes=False),
    )
    def k(table_hbm, idx_hbm, out_hbm, table_sh, idx_v, bufg0, bufg1,
          buft0, buft1, isem, gsem0, gsem1, osem0, osem1):
        sid = lax.axis_index("s")
        wid = sid * NC + lax.axis_index("c")
        ubase = wid * u_per_w
        bufg = (bufg0, bufg1)
        buft = (buft0, buft1)
        gsem = (gsem0, gsem1)
        osem = (osem0, osem1)

        @pl.when(sid == 0)
        def _():
            pltpu.sync_copy(table_hbm, table_sh)

        plsc.subcore_barrier()

        iota16 = lax.iota(jnp.int32, LANES)
        perm = tuple((iota16 + kk) & (LANES - 1) for kk in range(LANES))

        def iblk_load(i):
            pltpu.async_copy(
                idx_hbm.at[pl.ds((ubase + i * half) * BW, half * BW)],
                idx_v, isem)

        def iblk_wait():
            pltpu.make_async_copy(idx_hbm.at[pl.ds(0, half * BW)],
                                  idx_v, isem).wait()

        def gather_start(j, sg):
            pltpu.async_copy(
                table_sh.at[idx_v.at[pl.ds(j * BW, BW)]],
                bufg[sg], gsem[sg])

        def gather_wait(sg):
            pltpu.make_async_copy(table_sh.at[idx_v.at[pl.ds(0, BW)]],
                                  bufg[sg], gsem[sg]).wait()

        def transpose(s):
            g = bufg[s]
            t = buft[s]

            @plsc.parallel_loop(0, (D_MODEL // LANES) * (BW // LANES),
                                step=1, unroll=2)
            def tbody(bb):
                d0 = (bb & (D_MODEL // LANES - 1)) * LANES
                r0 = (bb >> 2) * LANES
                rows = r0 + iota16
                for kk in range(LANES):
                    cols = d0 + perm[kk]
                    vals = plsc.load_gather(g, [rows, cols])
                    plsc.store_scatter(t, [cols, rows], vals)

        def store_start(u, s):
            gu = ubase + u
            h = gu >> 7
            b0 = (gu & (BW - 1)) * BW
            pltpu.async_copy(buft[s], out_hbm.at[h, :, pl.ds(b0, BW)],
                             osem[s])

        def store_wait(s):
            pltpu.make_async_copy(buft[s], out_hbm.at[0, :, pl.ds(0, BW)],
                                  osem[s]).wait()

        def unit(u, sg, jn, first=False, start_next=True):
            gather_wait(sg)
            if start_next:
                gather_start(jn, sg ^ 1)
            if not first:
                store_wait(sg)
            transpose(sg)
            store_start(u, sg)

        iblk_load(0)
        iblk_wait()
        gather_start(0, 0)

        unit(0, 0, 1, first=True)
        unit(1, 1, 2, first=True)

        def body0(m, carry):
            u = 2 * m
            unit(u, 0, u + 1)
            unit(u + 1, 1, u + 2)
            return carry

        lax.fori_loop(1, half // 2 - 1, body0, 0)

        unit(half - 2, 0, half - 1)
        gather_wait(1)
        iblk_load(1)
        store_wait(1)
        transpose(1)
        store_start(half - 1, 1)
        iblk_wait()
        gather_start(0, 0)

        def body1(m, carry):
            u = 2 * m
            unit(half + u, 0, u + 1)
            unit(half + u + 1, 1, u + 2)
            return carry

        lax.fori_loop(0, half // 2 - 1, body1, 0)

        unit(2 * half - 2, 0, half - 1)
        unit(2 * half - 1, 1, 0, start_next=False)

        store_wait(0)
        store_wait(1)

    return k


def kernel(x, pe):
    nb, nh = x.shape
    V = pe.shape[0]
    H = V // 2
    idx = x.astype(jnp.int32)
    pe_sh = jnp.concatenate([pe[1:], jnp.zeros((1, D_MODEL), jnp.float32)], 0)
    table_m = jnp.concatenate([pe.reshape(H, 2 * D_MODEL),
                               pe_sh.reshape(H, 2 * D_MODEL)], 0)
    r = (idx >> 1) + (idx & 1) * H
    rt = r.T.reshape(nb * nh)
    out = _make_sc_gather(nb, nh, V)(table_m, rt)
    return jnp.transpose(out, (2, 0, 1))
